# Initial kernel scaffold; baseline (speedup 1.0000x reference)
import jax
import jax.numpy as jnp
from jax.experimental import pallas as pl


def kernel(data, W_init, b_init, W_bcl1, b_bcl1, W_bcl2, b_bcl2, W_bcl3, b_bcl3, W_bcl4, b_bcl4, W_f1, b_f1, W_f2, b_f2):
    raise NotImplementedError("write your pallas kernel here")


# speedup vs baseline: 3.1719x; 3.1719x over previous
import functools

import jax
import jax.numpy as jnp
from jax import lax
from jax.experimental import pallas as pl
from jax.experimental.pallas import tpu as pltpu
from jax.experimental.pallas import tpu_sc as plsc

R1 = 64 * 64 * 64
R2 = 32 * 32 * 32
HALF = R1 // 2
CPAD = 16


def _leaky(t):
    return jnp.where(t > 0.0, t, 0.1 * t)


def _dot11(a, b):
    return lax.dot_general(a, b, (((1,), (1,)), ((), ())),
                           preferred_element_type=jnp.float32)


def _prep_body(dT_ref, dfl_ref, wi_ref, bi_ref,
               f0_ref, one_ref, i1_ref, ilo_ref, ihi_ref, i2_ref, *, n_valid):
    x = dT_ref[0:1, :]
    y = dT_ref[1:2, :]
    z = dT_ref[2:3, :]
    cx = jnp.clip(jnp.floor(x * 64.0), 0.0, 63.0).astype(jnp.int32)
    cy = jnp.clip(jnp.floor(y * 64.0), 0.0, 63.0).astype(jnp.int32)
    cz = jnp.clip(jnp.floor(z * 64.0), 0.0, 63.0).astype(jnp.int32)
    idx1 = (cx * 64 + cy) * 64 + cz
    idx2 = ((cx >> 1) * 32 + (cy >> 1)) * 32 + (cz >> 1)
    col = lax.broadcasted_iota(jnp.int32, x.shape, 1)
    m = col < n_valid
    idx1 = jnp.where(m, idx1, 0)
    f0 = (wi_ref[0, 0] * dfl_ref[0:1, :] + wi_ref[0, 1] * dfl_ref[1:2, :]
          + wi_ref[0, 2] * dfl_ref[2:3, :] + bi_ref[0, 0])
    f0_ref[...] = jnp.where(m, f0, 0.0)
    one_ref[...] = jnp.where(m, 1.0, 0.0)
    i1_ref[...] = idx1
    ilo_ref[...] = jnp.where(idx1 < HALF, idx1, HALF)
    ihi_ref[...] = jnp.where(idx1 >= HALF, idx1 - HALF, HALF)
    i2_ref[...] = jnp.where(m, idx2, 0)


def _splat_body(vals_hbm, isc_hbm, zrows_hbm, out_hbm,
                idx_v, vals_v, shared, *, nr):
    c = lax.axis_index("c")
    s = lax.axis_index("s")
    npts = nr * 128
    for h in range(2):
        pltpu.sync_copy(zrows_hbm,
                        shared.at[pl.ds(s * (HALF // 16) + h * 4096, 4096)])
    pltpu.sync_copy(isc_hbm.at[c * 16 + s], idx_v)
    pltpu.sync_copy(vals_hbm.at[pl.ds(s * npts, npts)], vals_v)
    plsc.subcore_barrier()

    def body(j, carry):
        pltpu.sync_copy(vals_v.at[pl.ds(j * 128, 128)],
                        shared.at[idx_v.at[j]], add=True)
        return carry

    lax.fori_loop(0, nr, body, 0)
    plsc.subcore_barrier()
    for h in range(2):
        pltpu.sync_copy(shared.at[pl.ds(s * (HALF // 16) + h * 4096, 4096)],
                        out_hbm.at[c].at[pl.ds(s * (HALF // 16) + h * 4096, 4096)])


def _k3a_body(parts_ref, w1r_ref, b1_ref, wf1_ref, wf2_ref,
              proj1_ref, mp_ref, cp_ref):
    q = parts_ref[...]
    s = q[:, 0:1]
    cnt1 = q[:, 1:2]
    v1 = s / jnp.maximum(cnt1, 1.0)
    L1 = _leaky(v1 * w1r_ref[...] + b1_ref[...])

    We = lax.dot_general(wf2_ref[...], wf1_ref[...], (((1,), (0,)), ((), ())),
                         preferred_element_type=jnp.float32)
    proj1_ref[...] = _dot11(L1, We[:, 0:32])

    m = cnt1 * L1
    t = m.reshape(2048, 2, 32)
    a = t[:, 0, :] + t[:, 1, :]
    t = a.reshape(32, 2, 32, 32)
    mp = (t[:, 0] + t[:, 1]).reshape(1024, 32)
    t = cnt1.reshape(2048, 2, 1)
    a = t[:, 0, :] + t[:, 1, :]
    t = a.reshape(32, 2, 32, 1)
    cp = (t[:, 0] + t[:, 1]).reshape(1024, 1)
    mp_ref[...] = mp.reshape(1, 1024, 32)
    cp_ref[...] = cp.reshape(1, 1024, 1)


def _pool32(x):
    C = x.shape[1]
    t = x.reshape(1024, 2, C)
    a = t[:, 0, :] + t[:, 1, :]
    t = a.reshape(32, 2, 16, C)
    b = t[:, 0] + t[:, 1]
    r = b[0:16] + b[16:32]
    return r.reshape(256, C)


def _unpool16(p):
    C = p.shape[1]
    t = p.reshape(16, 16, C)
    t = jnp.broadcast_to(t[:, :, None, :], (16, 16, 2, C)).reshape(16, 32, C)
    t = jnp.broadcast_to(t[:, None, :, :], (16, 2, 32, C)).reshape(32, 32, C)
    t = jnp.broadcast_to(t[None, :, :, :], (2, 32, 32, C))
    return t.reshape(2048, C)


def _k3b_body(mp_ref, cp_ref, w2_ref, b2_ref, w3_ref, b3_ref, w4_ref, b4_ref,
              wf1_ref, bf1_ref, wf2_ref, bf2_ref, proj23_ref):
    lat2 = jnp.concatenate(
        [mp_ref[0] + mp_ref[1], mp_ref[2] + mp_ref[3]], axis=0)
    cnt2 = jnp.concatenate(
        [cp_ref[0] + cp_ref[1], cp_ref[2] + cp_ref[3]], axis=0)
    a2 = lat2 / jnp.maximum(cnt2, 1.0)
    L2 = _leaky(_dot11(a2, w2_ref[...]) + b2_ref[...])
    a3 = L2 * jnp.where(cnt2 > 0.0, 1.0, 0.0)
    L3 = _leaky(_dot11(a3, w3_ref[...]) + b3_ref[...])

    lat4 = _pool32(cnt2 * L3)
    cnt4 = _pool32(cnt2)
    a4 = lat4 / jnp.maximum(cnt4, 1.0)
    L4 = _leaky(_dot11(a4, w4_ref[...]) + b4_ref[...])

    We = lax.dot_general(wf2_ref[...], wf1_ref[...], (((1,), (0,)), ((), ())),
                         preferred_element_type=jnp.float32)
    be = _dot11(bf1_ref[...], wf2_ref[...]) + bf2_ref[...]
    u = _unpool16(_dot11(L4, We[:, 224:352]))
    proj23_ref[...] = (_dot11(L2, We[:, 32:96]) + _dot11(L3, We[:, 96:224])
                       + u + be)


def _slice_body(p1_hbm, p23_hbm, i1_hbm, i2_hbm, out_hbm,
                i1_v, i2_v, r1_v, r2_v, sem, *, nrows):
    c = lax.axis_index("c")
    s = lax.axis_index("s")
    npts = nrows * 128
    base = (c * 16 + s) * npts
    pltpu.sync_copy(i1_hbm.at[c * 16 + s], i1_v)
    pltpu.sync_copy(i2_hbm.at[c * 16 + s], i2_v)

    def gbody(j, carry):
        pltpu.async_copy(p1_hbm.at[i1_v.at[j]],
                         r1_v.at[pl.ds(j * 128, 128)], sem).wait()
        pltpu.async_copy(p23_hbm.at[i2_v.at[j]],
                         r2_v.at[pl.ds(j * 128, 128)], sem).wait()
        return carry

    lax.fori_loop(0, nrows, gbody, 0)

    def abody(k, carry):
        r1_v[k] = r1_v[k] + r2_v[k]
        return carry

    lax.fori_loop(0, npts, abody, 0)
    pltpu.sync_copy(r1_v, out_hbm.at[pl.ds(base, npts)])


def kernel(data, W_init, b_init, W_bcl1, b_bcl1, W_bcl2, b_bcl2, W_bcl3, b_bcl3,
           W_bcl4, b_bcl4, W_f1, b_f1, W_f2, b_f2):
    P = data.shape[0]
    P_pad = ((P + 4095) // 4096) * 4096
    nrows = P_pad // (32 * 128)
    nr2 = P_pad // (16 * 128)

    dT = jnp.pad(data.T, ((0, 0), (0, P_pad - P)))
    dfl = jnp.pad(data.reshape(3, P), ((0, 0), (0, P_pad - P)))

    f0r, oner, i1, ilo, ihi, i2 = pl.pallas_call(
        functools.partial(_prep_body, n_valid=P),
        out_shape=(
            jax.ShapeDtypeStruct((1, P_pad), jnp.float32),
            jax.ShapeDtypeStruct((1, P_pad), jnp.float32),
            jax.ShapeDtypeStruct((1, P_pad), jnp.int32),
            jax.ShapeDtypeStruct((1, P_pad), jnp.int32),
            jax.ShapeDtypeStruct((1, P_pad), jnp.int32),
            jax.ShapeDtypeStruct((1, P_pad), jnp.int32),
        ),
    )(dT, dfl, W_init, b_init.reshape(1, 1))

    vals = jnp.concatenate(
        [f0r, oner, jnp.zeros((6, P_pad), jnp.float32)], axis=0).T
    isc = jnp.concatenate([ilo.reshape(16, nr2, 128),
                           ihi.reshape(16, nr2, 128)], axis=0)
    i1_3d = i1.reshape(32, nrows, 128)
    i2_3d = i2.reshape(32, nrows, 128)
    zrows = jnp.zeros((4096, 8), jnp.float32)

    mesh = plsc.VectorSubcoreMesh(core_axis_name="c", subcore_axis_name="s")
    scparams = pltpu.CompilerParams(use_tc_tiling_on_sc=False)

    splat = functools.partial(
        pl.kernel,
        out_type=jax.ShapeDtypeStruct((2, HALF, 8), jnp.float32),
        mesh=mesh,
        compiler_params=scparams,
        scratch_types=[
            pltpu.VMEM((nr2, 128), jnp.int32),
            pltpu.VMEM((nr2 * 128, 8), jnp.float32),
            pltpu.VMEM_SHARED((HALF + 8, 8), jnp.float32),
        ],
    )(functools.partial(_splat_body, nr=nr2))
    parts = splat(vals, isc, zrows).reshape(R1, 8)

    wf2p = jnp.pad(W_f2, ((0, CPAD - W_f2.shape[0]), (0, 0)))
    bf2p = jnp.pad(b_f2, (0, CPAD - b_f2.shape[0])).reshape(1, CPAD)

    proj1, mp, cp = pl.pallas_call(
        _k3a_body,
        grid=(64,),
        in_specs=[
            pl.BlockSpec((4096, 8), lambda x: (x, 0)),
            pl.BlockSpec((1, 32), lambda x: (0, 0)),
            pl.BlockSpec((1, 32), lambda x: (0, 0)),
            pl.BlockSpec((32, 352), lambda x: (0, 0)),
            pl.BlockSpec((CPAD, 32), lambda x: (0, 0)),
        ],
        out_specs=(
            pl.BlockSpec((4096, CPAD), lambda x: (x, 0)),
            pl.BlockSpec((1, 1024, 32), lambda x: (x, 0, 0)),
            pl.BlockSpec((1, 1024, 1), lambda x: (x, 0, 0)),
        ),
        out_shape=(
            jax.ShapeDtypeStruct((R1, CPAD), jnp.float32),
            jax.ShapeDtypeStruct((64, 1024, 32), jnp.float32),
            jax.ShapeDtypeStruct((64, 1024, 1), jnp.float32),
        ),
    )(parts, W_bcl1.reshape(1, 32), b_bcl1.reshape(1, 32), W_f1, wf2p)

    (proj23,) = pl.pallas_call(
        _k3b_body,
        grid=(16,),
        in_specs=[
            pl.BlockSpec((4, 1024, 32), lambda i: (i, 0, 0)),
            pl.BlockSpec((4, 1024, 1), lambda i: (i, 0, 0)),
            pl.BlockSpec((64, 32), lambda i: (0, 0)),
            pl.BlockSpec((1, 64), lambda i: (0, 0)),
            pl.BlockSpec((128, 64), lambda i: (0, 0)),
            pl.BlockSpec((1, 128), lambda i: (0, 0)),
            pl.BlockSpec((128, 128), lambda i: (0, 0)),
            pl.BlockSpec((1, 128), lambda i: (0, 0)),
            pl.BlockSpec((32, 352), lambda i: (0, 0)),
            pl.BlockSpec((1, 32), lambda i: (0, 0)),
            pl.BlockSpec((CPAD, 32), lambda i: (0, 0)),
            pl.BlockSpec((1, CPAD), lambda i: (0, 0)),
        ],
        out_specs=(pl.BlockSpec((R2 // 16, CPAD), lambda i: (i, 0)),),
        out_shape=(jax.ShapeDtypeStruct((R2, CPAD), jnp.float32),),
    )(
        mp, cp,
        W_bcl2, b_bcl2.reshape(1, 64),
        W_bcl3, b_bcl3.reshape(1, 128),
        W_bcl4, b_bcl4.reshape(1, 128),
        W_f1, b_f1.reshape(1, 32), wf2p, bf2p,
    )

    slicer = functools.partial(
        pl.kernel,
        out_type=jax.ShapeDtypeStruct((P_pad, CPAD), jnp.float32),
        mesh=mesh,
        compiler_params=scparams,
        scratch_types=[
            pltpu.VMEM((nrows, 128), jnp.int32),
            pltpu.VMEM((nrows, 128), jnp.int32),
            pltpu.VMEM((nrows * 128, CPAD), jnp.float32),
            pltpu.VMEM((nrows * 128, CPAD), jnp.float32),
            pltpu.SemaphoreType.DMA,
        ],
    )(functools.partial(_slice_body, nrows=nrows))
    pts = slicer(proj1, proj23, i1_3d, i2_3d)

    return pts[:P, :13].T.reshape(1, 13, P, 1, 1)
